# Initial kernel scaffold; baseline (speedup 1.0000x reference)
import jax
import jax.numpy as jnp
from jax.experimental import pallas as pl


def kernel(x, states, params):
    raise NotImplementedError("write your pallas kernel here")


# speedup vs baseline: 3.8211x; 3.8211x over previous
import functools
import math

import jax
import jax.numpy as jnp
from jax import lax
from jax.experimental import pallas as pl
from jax.experimental.pallas import tpu as pltpu
from jax.experimental.pallas import tpu_sc as plsc

N = 4096
K = 16
F = 128
SD = 4
HP = 128
E = N * K
G4 = 512

NW = 32
CH = 128
NCH = E // (NW * CH)

_RB = 256
_DB = 512


def _knn_kernel(s_ref, sT_ref, idx_ref):
    i = pl.program_id(0)
    s_r = s_ref[...]
    sT = sT_ref[...]
    sq_r = jnp.sum(s_r * s_r, axis=1, keepdims=True)
    sq_all = jnp.sum(sT * sT, axis=0, keepdims=True)
    cross = jnp.dot(s_r, sT, preferred_element_type=jnp.float32)
    d2 = sq_r + sq_all - 2.0 * cross
    col = lax.broadcasted_iota(jnp.int32, (_RB, N), 1)
    row = i * _RB + lax.broadcasted_iota(jnp.int32, (_RB, N), 0)
    d2 = jnp.where(col == row, d2 + 1e9, d2)
    picks = []
    for _ in range(K):
        m = jnp.min(d2, axis=1, keepdims=True)
        cand = jnp.where(d2 == m, col, N)
        sel = jnp.min(cand, axis=1, keepdims=True)
        picks.append(sel)
        d2 = jnp.where(col == sel, jnp.float32(1e30), d2)
    idx_ref[...] = jnp.concatenate(picks, axis=1)


def _knn(states):
    sT = states.T
    return pl.pallas_call(
        _knn_kernel,
        grid=(N // _RB,),
        in_specs=[
            pl.BlockSpec((_RB, SD), lambda i: (i, 0)),
            pl.BlockSpec((SD, N), lambda i: (0, 0)),
        ],
        out_specs=pl.BlockSpec((_RB, K), lambda i: (i, 0)),
        out_shape=jax.ShapeDtypeStruct((N, K), jnp.int32),
    )(states, sT)


def _node_u_kernel(x_ref, s_ref, w1x_ref, w1s_ref, b1_ref, u_ref, t_ref):
    t = jnp.dot(s_ref[...], w1s_ref[...], preferred_element_type=jnp.float32)
    u = jnp.dot(x_ref[...], w1x_ref[...], preferred_element_type=jnp.float32)
    u_ref[...] = u + t + b1_ref[...]
    t_ref[...] = t


def _node_u(x, states, w1x, w1s, b1):
    return pl.pallas_call(
        _node_u_kernel,
        out_shape=(jax.ShapeDtypeStruct((N, HP), jnp.float32),
                   jax.ShapeDtypeStruct((N, HP), jnp.float32)),
    )(x, states, w1x, w1s, b1)


@functools.cache
def _make_sc_gather(D):
    mesh = plsc.VectorSubcoreMesh(core_axis_name="c", subcore_axis_name="s")

    @functools.partial(
        pl.kernel,
        out_type=jax.ShapeDtypeStruct((E, D), jnp.float32),
        mesh=mesh,
        scratch_types=[
            pltpu.VMEM((NCH, CH), jnp.int32),
            pltpu.VMEM((CH, D), jnp.float32),
            pltpu.SemaphoreType.DMA,
        ],
    )
    def gk(table_hbm, idx_hbm, out_hbm, idx_v, rows_v, sem):
        wid = lax.axis_index("s") * 2 + lax.axis_index("c")
        pltpu.sync_copy(idx_hbm.at[wid], idx_v)
        base = wid * (NCH * CH)

        def step(ci, carry):
            pltpu.async_copy(table_hbm.at[idx_v.at[ci]], rows_v, sem).wait()
            pltpu.sync_copy(rows_v, out_hbm.at[pl.ds(base + ci * CH, CH)])
            return carry

        lax.fori_loop(0, NCH, step, 0)

    return gk


def _gather128(table, idx3):
    return _make_sc_gather(F)(table, idx3)


def _edge_agg_kernel(ue_ref, t_ref, w2_ref, b2_ref, w3_ref, b3_ref,
                     wq_ref, bq_ref, wkT_ref, ws_ref, bs_ref,
                     agg_ref, r_ref, sb_ref):
    ue3 = ue_ref[...].reshape(_DB, K, HP)
    t3 = t_ref[...].reshape(_DB, 1, HP)
    h1 = jnp.maximum(ue3 - t3, 0.0).reshape(_DB * K, HP)
    h2 = jnp.maximum(
        jnp.dot(h1, w2_ref[...], preferred_element_type=jnp.float32)
        + b2_ref[...], 0.0)
    agg20 = jnp.sum(h2.reshape(_DB, K, HP), axis=1)
    agg = (jnp.dot(agg20, w3_ref[...], preferred_element_type=jnp.float32)
           + jnp.float32(K) * b3_ref[...])
    q = jnp.dot(agg, wq_ref[...], preferred_element_type=jnp.float32) + bq_ref[...]
    r_ref[...] = jnp.dot(q, wkT_ref[...],
                         preferred_element_type=jnp.float32) / math.sqrt(float(F))
    sb_ref[...] = (jnp.dot(agg, ws_ref[...], preferred_element_type=jnp.float32)
                   + bs_ref[...])
    agg_ref[...] = agg


def _edge_agg(ue, t, w2, b2, w3, b3, wq, bq, wkT, ws, bs):
    nb = N // _DB
    full = lambda shape: pl.BlockSpec(shape, lambda i: (0, 0))
    return pl.pallas_call(
        _edge_agg_kernel,
        grid=(nb,),
        in_specs=[
            pl.BlockSpec((_DB * K, HP), lambda i: (i, 0)),
            pl.BlockSpec((_DB, HP), lambda i: (i, 0)),
            full((HP, HP)), full((1, HP)), full((HP, F)), full((1, F)),
            full((F, F)), full((1, F)), full((F, F)),
            full((F, F)), full((1, F)),
        ],
        out_specs=(pl.BlockSpec((_DB, F), lambda i: (i, 0)),
                   pl.BlockSpec((_DB, F), lambda i: (i, 0)),
                   pl.BlockSpec((_DB, F), lambda i: (i, 0))),
        out_shape=(jax.ShapeDtypeStruct((N, F), jnp.float32),
                   jax.ShapeDtypeStruct((N, F), jnp.float32),
                   jax.ShapeDtypeStruct((N, F), jnp.float32)),
    )(ue, t, w2, b2, w3, b3, wq, bq, wkT, ws, bs)


def _attn_pre_kernel(g_ref, r_ref, sb_ref, x_ref, wv_ref, bv_ref,
                     wa_ref, wb_ref, b0_ref, pre_ref):
    g3 = g_ref[...].reshape(_DB, K, F)
    r3 = r_ref[...].reshape(_DB, 1, F)
    al = jnp.sum(g3 * r3, axis=2)
    amax = jnp.max(al, axis=1, keepdims=True)
    ex = jnp.exp(al - amax)
    den = jnp.sum(ex, axis=1, keepdims=True)
    w = ex / (den + 1e-16)
    msum = jnp.sum(g3 * w[:, :, None], axis=1)
    out = (jnp.dot(msum, wv_ref[...], preferred_element_type=jnp.float32)
           + bv_ref[...] + sb_ref[...])
    pre_ref[...] = (
        jnp.dot(out, wa_ref[...], preferred_element_type=jnp.float32)
        + jnp.dot(x_ref[...], wb_ref[...], preferred_element_type=jnp.float32)
        + b0_ref[...])


def _attn_pre(g, r, sb, x, wv, bv, wa, wb, b0):
    nb = N // _DB
    full = lambda shape: pl.BlockSpec(shape, lambda i: (0, 0))
    return pl.pallas_call(
        _attn_pre_kernel,
        grid=(nb,),
        in_specs=[
            pl.BlockSpec((_DB * K, F), lambda i: (i, 0)),
            pl.BlockSpec((_DB, F), lambda i: (i, 0)),
            pl.BlockSpec((_DB, F), lambda i: (i, 0)),
            pl.BlockSpec((_DB, F), lambda i: (i, 0)),
            full((F, F)), full((1, F)),
            full((F, G4)), full((F, G4)), full((1, G4)),
        ],
        out_specs=pl.BlockSpec((_DB, G4), lambda i: (i, 0)),
        out_shape=jax.ShapeDtypeStruct((N, G4), jnp.float32),
    )(g, r, sb, x, wv, bv, wa, wb, b0)


def _lstm_kernel(pre_ref, w0_ref, wi1_ref, wh1_ref, b1_ref, out_ref, *, relu):
    w0 = w0_ref[...]
    wi1 = wi1_ref[...]
    wh1 = wh1_ref[...]
    b1 = b1_ref[...]

    def gates(g, c):
        i = jax.nn.sigmoid(g[:, 0:F])
        f = jax.nn.sigmoid(g[:, F:2 * F])
        gg = jnp.tanh(g[:, 2 * F:3 * F])
        o = jax.nn.sigmoid(g[:, 3 * F:])
        cn = f * c + i * gg
        hn = o * jnp.tanh(cn)
        return hn, cn

    def step(t, carry):
        h0, c0, h1, c1 = carry
        g0 = pre_ref[pl.ds(t, 1), :] + jnp.dot(
            h0, w0, preferred_element_type=jnp.float32)
        h0n, c0n = gates(g0, c0)
        g1 = (jnp.dot(h0n, wi1, preferred_element_type=jnp.float32)
              + jnp.dot(h1, wh1, preferred_element_type=jnp.float32) + b1)
        h1n, c1n = gates(g1, c1)
        out_ref[pl.ds(t, 1), :] = jnp.maximum(h1n, 0.0) if relu else h1n
        return (h0n, c0n, h1n, c1n)

    z = jnp.zeros((1, F), jnp.float32)
    lax.fori_loop(0, N, step, (z, z, z, z))


def _lstm(pre, w0, wi1, wh1, b1, relu):
    return pl.pallas_call(
        functools.partial(_lstm_kernel, relu=relu),
        out_shape=jax.ShapeDtypeStruct((N, F), jnp.float32),
    )(pre, w0, wi1, wh1, b1)


def _head_kernel(h_ref, w_ref, b_ref, out_ref):
    hr = jnp.maximum(h_ref[...], 0.0)
    out_ref[...] = (jnp.dot(hr, w_ref[...], preferred_element_type=jnp.float32)
                    + b_ref[...])


def _head(h, w, b):
    return pl.pallas_call(
        _head_kernel,
        out_shape=jax.ShapeDtypeStruct((N, 1), jnp.float32),
    )(h, w, b.reshape(1, 1))


def _pad_cols(a, width):
    return jnp.pad(a, [(0, 0)] * (a.ndim - 1) + [(0, width - a.shape[-1])])


def kernel(x, states, params):
    p = params
    idx = _knn(states)
    idx3 = idx.reshape(NW, NCH, CH)

    xc = x
    for li in range(4):
        w1 = p['phi_W1'][li]
        w1x = _pad_cols(w1[:F], HP)
        w1s = _pad_cols(w1[F:], HP)
        b1 = _pad_cols(p['phi_b1'][li], HP).reshape(1, HP)
        w2 = jnp.pad(p['phi_W2'][li], ((0, HP - 20), (0, HP - 20)))
        b2 = _pad_cols(p['phi_b2'][li], HP).reshape(1, HP)
        w3 = jnp.pad(p['phi_W3'][li], ((0, HP - 20), (0, 0)))
        b3 = p['phi_b3'][li].reshape(1, F)

        u, t = _node_u(xc, states, w1x, w1s, b1)
        ue = _gather128(u, idx3)
        agg, r, sb = _edge_agg(
            ue, t, w2, b2, w3, b3,
            p['tc_Wq'][li], p['tc_bq'][li].reshape(1, F),
            p['tc_Wk'][li].T,
            p['tc_Ws'][li], p['tc_bs'][li].reshape(1, F))
        g = _gather128(agg, idx3)

        wih0 = p['lstm_Wih0'][li]
        b0 = (p['lstm_bih0'][li] + p['lstm_bhh0'][li]).reshape(1, G4)
        pre = _attn_pre(
            g, r, sb, xc,
            p['tc_Wv'][li], p['tc_bv'][li].reshape(1, F),
            wih0[:, :F].T, wih0[:, F:].T, b0)

        b1s = (p['lstm_bih1'][li] + p['lstm_bhh1'][li]).reshape(1, G4)
        xc = _lstm(pre,
                   p['lstm_Whh0'][li].T,
                   p['lstm_Wih1'][li].T,
                   p['lstm_Whh1'][li].T,
                   b1s,
                   relu=(li in (0, 2)))

    h4 = xc
    h = _head(h4, p['cbf_W'], p['cbf_b'])
    return h4, h


# speedup vs baseline: 5.4559x; 1.4278x over previous
import functools
import math

import jax
import jax.numpy as jnp
from jax import lax
from jax.experimental import pallas as pl
from jax.experimental.pallas import tpu as pltpu
from jax.experimental.pallas import tpu_sc as plsc

N = 4096
K = 16
F = 128
SD = 4
HP = 128
E = N * K
G4 = 512

NW = 32
CH = 128
NCH = E // (NW * CH)

_RB = 256
_DB = 512


def _knn_kernel(s_ref, sT_ref, idx_ref):
    i = pl.program_id(0)
    s_r = s_ref[...]
    sT = sT_ref[...]
    sq_r = jnp.sum(s_r * s_r, axis=1, keepdims=True)
    sq_all = jnp.sum(sT * sT, axis=0, keepdims=True)
    cross = jnp.dot(s_r, sT, preferred_element_type=jnp.float32)
    d2 = sq_r + sq_all - 2.0 * cross
    col = lax.broadcasted_iota(jnp.int32, (_RB, N), 1)
    row = i * _RB + lax.broadcasted_iota(jnp.int32, (_RB, N), 0)
    d2 = jnp.where(col == row, d2 + 1e9, d2)
    picks = []
    for _ in range(K):
        m = jnp.min(d2, axis=1, keepdims=True)
        cand = jnp.where(d2 == m, col, N)
        sel = jnp.min(cand, axis=1, keepdims=True)
        picks.append(sel)
        d2 = jnp.where(col == sel, jnp.float32(1e30), d2)
    idx_ref[...] = jnp.concatenate(picks, axis=1)


def _knn(states):
    sT = states.T
    return pl.pallas_call(
        _knn_kernel,
        grid=(N // _RB,),
        in_specs=[
            pl.BlockSpec((_RB, SD), lambda i: (i, 0)),
            pl.BlockSpec((SD, N), lambda i: (0, 0)),
        ],
        out_specs=pl.BlockSpec((_RB, K), lambda i: (i, 0)),
        out_shape=jax.ShapeDtypeStruct((N, K), jnp.int32),
    )(states, sT)


def _node_u_kernel(x_ref, s_ref, w1x_ref, w1s_ref, b1_ref, u_ref, t_ref):
    t = jnp.dot(s_ref[...], w1s_ref[...], preferred_element_type=jnp.float32)
    u = jnp.dot(x_ref[...], w1x_ref[...], preferred_element_type=jnp.float32)
    u_ref[...] = u + t + b1_ref[...]
    t_ref[...] = t


def _node_u(x, states, w1x, w1s, b1):
    return pl.pallas_call(
        _node_u_kernel,
        out_shape=(jax.ShapeDtypeStruct((N, HP), jnp.float32),
                   jax.ShapeDtypeStruct((N, HP), jnp.float32)),
    )(x, states, w1x, w1s, b1)


@functools.cache
def _make_sc_gather(D):
    mesh = plsc.VectorSubcoreMesh(core_axis_name="c", subcore_axis_name="s")

    @functools.partial(
        pl.kernel,
        out_type=jax.ShapeDtypeStruct((E, D), jnp.float32),
        mesh=mesh,
        scratch_types=[
            pltpu.VMEM((NCH, CH), jnp.int32),
            pltpu.VMEM((CH, D), jnp.float32),
            pltpu.SemaphoreType.DMA,
        ],
    )
    def gk(table_hbm, idx_hbm, out_hbm, idx_v, rows_v, sem):
        wid = lax.axis_index("s") * 2 + lax.axis_index("c")
        pltpu.sync_copy(idx_hbm.at[wid], idx_v)
        base = wid * (NCH * CH)

        def step(ci, carry):
            pltpu.async_copy(table_hbm.at[idx_v.at[ci]], rows_v, sem).wait()
            pltpu.sync_copy(rows_v, out_hbm.at[pl.ds(base + ci * CH, CH)])
            return carry

        lax.fori_loop(0, NCH, step, 0)

    return gk


def _gather128(table, idx3):
    return _make_sc_gather(F)(table, idx3)


def _edge_agg_kernel(ue_ref, t_ref, w2_ref, b2_ref, w3_ref, b3_ref,
                     wq_ref, bq_ref, wkT_ref, ws_ref, bs_ref,
                     agg_ref, r_ref, sb_ref):
    ue3 = ue_ref[...].reshape(_DB, K, HP)
    t3 = t_ref[...].reshape(_DB, 1, HP)
    h1 = jnp.maximum(ue3 - t3, 0.0).reshape(_DB * K, HP)
    h2 = jnp.maximum(
        jnp.dot(h1, w2_ref[...], preferred_element_type=jnp.float32)
        + b2_ref[...], 0.0)
    agg20 = jnp.sum(h2.reshape(_DB, K, HP), axis=1)
    agg = (jnp.dot(agg20, w3_ref[...], preferred_element_type=jnp.float32)
           + jnp.float32(K) * b3_ref[...])
    q = jnp.dot(agg, wq_ref[...], preferred_element_type=jnp.float32) + bq_ref[...]
    r_ref[...] = jnp.dot(q, wkT_ref[...],
                         preferred_element_type=jnp.float32) / math.sqrt(float(F))
    sb_ref[...] = (jnp.dot(agg, ws_ref[...], preferred_element_type=jnp.float32)
                   + bs_ref[...])
    agg_ref[...] = agg


def _edge_agg(ue, t, w2, b2, w3, b3, wq, bq, wkT, ws, bs):
    nb = N // _DB
    full = lambda shape: pl.BlockSpec(shape, lambda i: (0, 0))
    return pl.pallas_call(
        _edge_agg_kernel,
        grid=(nb,),
        in_specs=[
            pl.BlockSpec((_DB * K, HP), lambda i: (i, 0)),
            pl.BlockSpec((_DB, HP), lambda i: (i, 0)),
            full((HP, HP)), full((1, HP)), full((HP, F)), full((1, F)),
            full((F, F)), full((1, F)), full((F, F)),
            full((F, F)), full((1, F)),
        ],
        out_specs=(pl.BlockSpec((_DB, F), lambda i: (i, 0)),
                   pl.BlockSpec((_DB, F), lambda i: (i, 0)),
                   pl.BlockSpec((_DB, F), lambda i: (i, 0))),
        out_shape=(jax.ShapeDtypeStruct((N, F), jnp.float32),
                   jax.ShapeDtypeStruct((N, F), jnp.float32),
                   jax.ShapeDtypeStruct((N, F), jnp.float32)),
    )(ue, t, w2, b2, w3, b3, wq, bq, wkT, ws, bs)


def _attn_pre_kernel(g_ref, r_ref, sb_ref, x_ref, wv_ref, bv_ref,
                     wa_ref, wb_ref, b0_ref, pre_ref):
    g3 = g_ref[...].reshape(_DB, K, F)
    r3 = r_ref[...].reshape(_DB, 1, F)
    al = jnp.sum(g3 * r3, axis=2)
    amax = jnp.max(al, axis=1, keepdims=True)
    ex = jnp.exp(al - amax)
    den = jnp.sum(ex, axis=1, keepdims=True)
    w = ex / (den + 1e-16)
    msum = jnp.sum(g3 * w[:, :, None], axis=1)
    out = (jnp.dot(msum, wv_ref[...], preferred_element_type=jnp.float32)
           + bv_ref[...] + sb_ref[...])
    pre_ref[...] = (
        jnp.dot(out, wa_ref[...], preferred_element_type=jnp.float32)
        + jnp.dot(x_ref[...], wb_ref[...], preferred_element_type=jnp.float32)
        + b0_ref[...])


def _attn_pre(g, r, sb, x, wv, bv, wa, wb, b0):
    nb = N // _DB
    full = lambda shape: pl.BlockSpec(shape, lambda i: (0, 0))
    return pl.pallas_call(
        _attn_pre_kernel,
        grid=(nb,),
        in_specs=[
            pl.BlockSpec((_DB * K, F), lambda i: (i, 0)),
            pl.BlockSpec((_DB, F), lambda i: (i, 0)),
            pl.BlockSpec((_DB, F), lambda i: (i, 0)),
            pl.BlockSpec((_DB, F), lambda i: (i, 0)),
            full((F, F)), full((1, F)),
            full((F, G4)), full((F, G4)), full((1, G4)),
        ],
        out_specs=pl.BlockSpec((_DB, G4), lambda i: (i, 0)),
        out_shape=jax.ShapeDtypeStruct((N, G4), jnp.float32),
    )(g, r, sb, x, wv, bv, wa, wb, b0)


def _lstm_kernel(pre_ref, w0_ref, wi1_ref, wh1_ref, b1_ref, out_ref, *, relu):
    w0 = w0_ref[...]
    wi1 = wi1_ref[...]
    wh1 = wh1_ref[...]
    b1 = b1_ref[...]

    def gates(g, c):
        i = jax.nn.sigmoid(g[:, 0:F])
        f = jax.nn.sigmoid(g[:, F:2 * F])
        gg = jnp.tanh(g[:, 2 * F:3 * F])
        o = jax.nn.sigmoid(g[:, 3 * F:])
        cn = f * c + i * gg
        hn = o * jnp.tanh(cn)
        return hn, cn

    def out_store(t, h):
        out_ref[pl.ds(t, 1), :] = jnp.maximum(h, 0.0) if relu else h

    def lay1(h0, h1, c1):
        g1 = (jnp.dot(h0, wi1, preferred_element_type=jnp.float32)
              + jnp.dot(h1, wh1, preferred_element_type=jnp.float32) + b1)
        return gates(g1, c1)

    z = jnp.zeros((1, F), jnp.float32)
    h0, c0 = gates(pre_ref[pl.ds(0, 1), :], z)

    def step(t, carry):
        h0, c0, h1, c1 = carry
        g0 = pre_ref[pl.ds(t, 1), :] + jnp.dot(
            h0, w0, preferred_element_type=jnp.float32)
        h1n, c1n = lay1(h0, h1, c1)
        h0n, c0n = gates(g0, c0)
        out_store(t - 1, h1n)
        return (h0n, c0n, h1n, c1n)

    h0, c0, h1, c1 = lax.fori_loop(1, N, step, (h0, c0, z, z))
    h1n, _ = lay1(h0, h1, c1)
    out_store(N - 1, h1n)


def _lstm(pre, w0, wi1, wh1, b1, relu):
    return pl.pallas_call(
        functools.partial(_lstm_kernel, relu=relu),
        out_shape=jax.ShapeDtypeStruct((N, F), jnp.float32),
    )(pre, w0, wi1, wh1, b1)


def _head_kernel(h_ref, w_ref, b_ref, out_ref):
    hr = jnp.maximum(h_ref[...], 0.0)
    out_ref[...] = (jnp.dot(hr, w_ref[...], preferred_element_type=jnp.float32)
                    + b_ref[...])


def _head(h, w, b):
    return pl.pallas_call(
        _head_kernel,
        out_shape=jax.ShapeDtypeStruct((N, 1), jnp.float32),
    )(h, w, b.reshape(1, 1))


def _pad_cols(a, width):
    return jnp.pad(a, [(0, 0)] * (a.ndim - 1) + [(0, width - a.shape[-1])])


def kernel(x, states, params):
    p = params
    idx = _knn(states)
    idx3 = idx.reshape(NW, NCH, CH)

    xc = x
    for li in range(4):
        w1 = p['phi_W1'][li]
        w1x = _pad_cols(w1[:F], HP)
        w1s = _pad_cols(w1[F:], HP)
        b1 = _pad_cols(p['phi_b1'][li], HP).reshape(1, HP)
        w2 = jnp.pad(p['phi_W2'][li], ((0, HP - 20), (0, HP - 20)))
        b2 = _pad_cols(p['phi_b2'][li], HP).reshape(1, HP)
        w3 = jnp.pad(p['phi_W3'][li], ((0, HP - 20), (0, 0)))
        b3 = p['phi_b3'][li].reshape(1, F)

        u, t = _node_u(xc, states, w1x, w1s, b1)
        ue = _gather128(u, idx3)
        agg, r, sb = _edge_agg(
            ue, t, w2, b2, w3, b3,
            p['tc_Wq'][li], p['tc_bq'][li].reshape(1, F),
            p['tc_Wk'][li].T,
            p['tc_Ws'][li], p['tc_bs'][li].reshape(1, F))
        g = _gather128(agg, idx3)

        wih0 = p['lstm_Wih0'][li]
        b0 = (p['lstm_bih0'][li] + p['lstm_bhh0'][li]).reshape(1, G4)
        pre = _attn_pre(
            g, r, sb, xc,
            p['tc_Wv'][li], p['tc_bv'][li].reshape(1, F),
            wih0[:, :F].T, wih0[:, F:].T, b0)

        b1s = (p['lstm_bih1'][li] + p['lstm_bhh1'][li]).reshape(1, G4)
        xc = _lstm(pre,
                   p['lstm_Whh0'][li].T,
                   p['lstm_Wih1'][li].T,
                   p['lstm_Whh1'][li].T,
                   b1s,
                   relu=(li in (0, 2)))

    h4 = xc
    h = _head(h4, p['cbf_W'], p['cbf_b'])
    return h4, h


# speedup vs baseline: 6.1975x; 1.1359x over previous
import functools
import math

import jax
import jax.numpy as jnp
from jax import lax
from jax.experimental import pallas as pl
from jax.experimental.pallas import tpu as pltpu
from jax.experimental.pallas import tpu_sc as plsc

N = 4096
K = 16
F = 128
SD = 4
HP = 128
E = N * K
G4 = 512

NW = 32
CH = 128
NCH = E // (NW * CH)

_RB = 256
_DB = 512


def _knn_kernel(s_ref, sT_ref, idx_ref):
    i = pl.program_id(0)
    s_r = s_ref[...]
    sT = sT_ref[...]
    sq_r = jnp.sum(s_r * s_r, axis=1, keepdims=True)
    sq_all = jnp.sum(sT * sT, axis=0, keepdims=True)
    cross = jnp.dot(s_r, sT, preferred_element_type=jnp.float32)
    d2 = sq_r + sq_all - 2.0 * cross
    col = lax.broadcasted_iota(jnp.int32, (_RB, N), 1)
    row = i * _RB + lax.broadcasted_iota(jnp.int32, (_RB, N), 0)
    d2 = jnp.where(col == row, d2 + 1e9, d2)
    picks = []
    for _ in range(K):
        m = jnp.min(d2, axis=1, keepdims=True)
        cand = jnp.where(d2 == m, col, N)
        sel = jnp.min(cand, axis=1, keepdims=True)
        picks.append(sel)
        d2 = jnp.where(col == sel, jnp.float32(1e30), d2)
    idx_ref[...] = jnp.concatenate(picks, axis=1)


def _knn(states):
    sT = states.T
    return pl.pallas_call(
        _knn_kernel,
        grid=(N // _RB,),
        in_specs=[
            pl.BlockSpec((_RB, SD), lambda i: (i, 0)),
            pl.BlockSpec((SD, N), lambda i: (0, 0)),
        ],
        out_specs=pl.BlockSpec((_RB, K), lambda i: (i, 0)),
        out_shape=jax.ShapeDtypeStruct((N, K), jnp.int32),
    )(states, sT)


def _node_u_kernel(x_ref, s_ref, w1x_ref, w1s_ref, b1_ref, u_ref, t_ref):
    t = jnp.dot(s_ref[...], w1s_ref[...], preferred_element_type=jnp.float32)
    u = jnp.dot(x_ref[...], w1x_ref[...], preferred_element_type=jnp.float32)
    u_ref[...] = u + t + b1_ref[...]
    t_ref[...] = t


def _node_u(x, states, w1x, w1s, b1):
    return pl.pallas_call(
        _node_u_kernel,
        out_shape=(jax.ShapeDtypeStruct((N, HP), jnp.float32),
                   jax.ShapeDtypeStruct((N, HP), jnp.float32)),
    )(x, states, w1x, w1s, b1)


@functools.cache
def _make_sc_gather(D):
    mesh = plsc.VectorSubcoreMesh(core_axis_name="c", subcore_axis_name="s")

    @functools.partial(
        pl.kernel,
        out_type=jax.ShapeDtypeStruct((E, D), jnp.float32),
        mesh=mesh,
        scratch_types=[
            pltpu.VMEM((NCH, CH), jnp.int32),
            pltpu.VMEM((CH, D), jnp.float32),
            pltpu.SemaphoreType.DMA,
        ],
    )
    def gk(table_hbm, idx_hbm, out_hbm, idx_v, rows_v, sem):
        wid = lax.axis_index("s") * 2 + lax.axis_index("c")
        pltpu.sync_copy(idx_hbm.at[wid], idx_v)
        base = wid * (NCH * CH)

        def step(ci, carry):
            pltpu.async_copy(table_hbm.at[idx_v.at[ci]], rows_v, sem).wait()
            pltpu.sync_copy(rows_v, out_hbm.at[pl.ds(base + ci * CH, CH)])
            return carry

        lax.fori_loop(0, NCH, step, 0)

    return gk


def _gather128(table, idx3):
    return _make_sc_gather(F)(table, idx3)


def _edge_agg_kernel(ue_ref, t_ref, w2_ref, b2_ref, w3_ref, b3_ref,
                     wq_ref, bq_ref, wkT_ref, ws_ref, bs_ref,
                     agg_ref, r_ref, sb_ref):
    ue3 = ue_ref[...].reshape(_DB, K, HP)
    t3 = t_ref[...].reshape(_DB, 1, HP)
    h1 = jnp.maximum(ue3 - t3, 0.0).reshape(_DB * K, HP)
    h2 = jnp.maximum(
        jnp.dot(h1, w2_ref[...], preferred_element_type=jnp.float32)
        + b2_ref[...], 0.0)
    agg20 = jnp.sum(h2.reshape(_DB, K, HP), axis=1)
    agg = (jnp.dot(agg20, w3_ref[...], preferred_element_type=jnp.float32)
           + jnp.float32(K) * b3_ref[...])
    q = jnp.dot(agg, wq_ref[...], preferred_element_type=jnp.float32) + bq_ref[...]
    r_ref[...] = jnp.dot(q, wkT_ref[...],
                         preferred_element_type=jnp.float32) / math.sqrt(float(F))
    sb_ref[...] = (jnp.dot(agg, ws_ref[...], preferred_element_type=jnp.float32)
                   + bs_ref[...])
    agg_ref[...] = agg


def _edge_agg(ue, t, w2, b2, w3, b3, wq, bq, wkT, ws, bs):
    nb = N // _DB
    full = lambda shape: pl.BlockSpec(shape, lambda i: (0, 0))
    return pl.pallas_call(
        _edge_agg_kernel,
        grid=(nb,),
        in_specs=[
            pl.BlockSpec((_DB * K, HP), lambda i: (i, 0)),
            pl.BlockSpec((_DB, HP), lambda i: (i, 0)),
            full((HP, HP)), full((1, HP)), full((HP, F)), full((1, F)),
            full((F, F)), full((1, F)), full((F, F)),
            full((F, F)), full((1, F)),
        ],
        out_specs=(pl.BlockSpec((_DB, F), lambda i: (i, 0)),
                   pl.BlockSpec((_DB, F), lambda i: (i, 0)),
                   pl.BlockSpec((_DB, F), lambda i: (i, 0))),
        out_shape=(jax.ShapeDtypeStruct((N, F), jnp.float32),
                   jax.ShapeDtypeStruct((N, F), jnp.float32),
                   jax.ShapeDtypeStruct((N, F), jnp.float32)),
    )(ue, t, w2, b2, w3, b3, wq, bq, wkT, ws, bs)


def _attn_pre_kernel(g_ref, r_ref, sb_ref, x_ref, wv_ref, bv_ref,
                     wa_ref, wb_ref, b0_ref, pre_ref):
    g3 = g_ref[...].reshape(_DB, K, F)
    r3 = r_ref[...].reshape(_DB, 1, F)
    al = jnp.sum(g3 * r3, axis=2)
    amax = jnp.max(al, axis=1, keepdims=True)
    ex = jnp.exp(al - amax)
    den = jnp.sum(ex, axis=1, keepdims=True)
    w = ex / (den + 1e-16)
    msum = jnp.sum(g3 * w[:, :, None], axis=1)
    out = (jnp.dot(msum, wv_ref[...], preferred_element_type=jnp.float32)
           + bv_ref[...] + sb_ref[...])
    pre_ref[...] = (
        jnp.dot(out, wa_ref[...], preferred_element_type=jnp.float32)
        + jnp.dot(x_ref[...], wb_ref[...], preferred_element_type=jnp.float32)
        + b0_ref[...])


def _attn_pre(g, r, sb, x, wv, bv, wa, wb, b0):
    nb = N // _DB
    full = lambda shape: pl.BlockSpec(shape, lambda i: (0, 0))
    return pl.pallas_call(
        _attn_pre_kernel,
        grid=(nb,),
        in_specs=[
            pl.BlockSpec((_DB * K, F), lambda i: (i, 0)),
            pl.BlockSpec((_DB, F), lambda i: (i, 0)),
            pl.BlockSpec((_DB, F), lambda i: (i, 0)),
            pl.BlockSpec((_DB, F), lambda i: (i, 0)),
            full((F, F)), full((1, F)),
            full((F, G4)), full((F, G4)), full((1, G4)),
        ],
        out_specs=pl.BlockSpec((_DB, G4), lambda i: (i, 0)),
        out_shape=jax.ShapeDtypeStruct((N, G4), jnp.float32),
    )(g, r, sb, x, wv, bv, wa, wb, b0)


def _lstm_kernel(pre_ref, w0_ref, wc_ref, b1_ref, out_ref, *, relu):
    w0 = w0_ref[...].astype(jnp.bfloat16)
    wc = wc_ref[...].astype(jnp.bfloat16)
    b1 = b1_ref[...]
    def gates(g, c):
        i = jax.nn.sigmoid(g[:, 0:F])
        f = jax.nn.sigmoid(g[:, F:2 * F])
        gg = jnp.tanh(g[:, 2 * F:3 * F])
        o = jax.nn.sigmoid(g[:, 3 * F:])
        cn = f * c + i * gg
        hn = o * jnp.tanh(cn)
        return hn, cn

    def out_store(t, h):
        out_ref[pl.ds(t, 1), :] = jnp.maximum(h, 0.0) if relu else h

    def lay1(h0, h1, c1):
        hcat = jnp.concatenate([h0, h1], axis=1).astype(jnp.bfloat16)
        g1 = jnp.dot(hcat, wc, preferred_element_type=jnp.float32,
                     precision=lax.Precision.DEFAULT) + b1
        return gates(g1, c1)

    z = jnp.zeros((1, F), jnp.float32)
    h0, c0 = gates(pre_ref[pl.ds(0, 1), :], z)

    def step(t, carry):
        h0, c0, h1, c1 = carry
        g0 = pre_ref[pl.ds(t, 1), :] + jnp.dot(
            h0.astype(jnp.bfloat16), w0, preferred_element_type=jnp.float32,
            precision=lax.Precision.DEFAULT)
        h1n, c1n = lay1(h0, h1, c1)
        h0n, c0n = gates(g0, c0)
        out_store(t - 1, h1n)
        return (h0n, c0n, h1n, c1n)

    h0, c0, h1, c1 = lax.fori_loop(1, N, step, (h0, c0, z, z), unroll=2)
    h1n, _ = lay1(h0, h1, c1)
    out_store(N - 1, h1n)


def _lstm(pre, w0, wc, b1, relu):
    return pl.pallas_call(
        functools.partial(_lstm_kernel, relu=relu),
        out_shape=jax.ShapeDtypeStruct((N, F), jnp.float32),
    )(pre, w0, wc, b1)


def _head_kernel(h_ref, w_ref, b_ref, out_ref):
    hr = jnp.maximum(h_ref[...], 0.0)
    out_ref[...] = (jnp.dot(hr, w_ref[...], preferred_element_type=jnp.float32)
                    + b_ref[...])


def _head(h, w, b):
    return pl.pallas_call(
        _head_kernel,
        out_shape=jax.ShapeDtypeStruct((N, 1), jnp.float32),
    )(h, w, b.reshape(1, 1))


def _pad_cols(a, width):
    return jnp.pad(a, [(0, 0)] * (a.ndim - 1) + [(0, width - a.shape[-1])])


def kernel(x, states, params):
    p = params
    idx = _knn(states)
    idx3 = idx.reshape(NW, NCH, CH)

    xc = x
    for li in range(4):
        w1 = p['phi_W1'][li]
        w1x = _pad_cols(w1[:F], HP)
        w1s = _pad_cols(w1[F:], HP)
        b1 = _pad_cols(p['phi_b1'][li], HP).reshape(1, HP)
        w2 = jnp.pad(p['phi_W2'][li], ((0, HP - 20), (0, HP - 20)))
        b2 = _pad_cols(p['phi_b2'][li], HP).reshape(1, HP)
        w3 = jnp.pad(p['phi_W3'][li], ((0, HP - 20), (0, 0)))
        b3 = p['phi_b3'][li].reshape(1, F)

        u, t = _node_u(xc, states, w1x, w1s, b1)
        ue = _gather128(u, idx3)
        agg, r, sb = _edge_agg(
            ue, t, w2, b2, w3, b3,
            p['tc_Wq'][li], p['tc_bq'][li].reshape(1, F),
            p['tc_Wk'][li].T,
            p['tc_Ws'][li], p['tc_bs'][li].reshape(1, F))
        g = _gather128(agg, idx3)

        wih0 = p['lstm_Wih0'][li]
        b0 = (p['lstm_bih0'][li] + p['lstm_bhh0'][li]).reshape(1, G4)
        pre = _attn_pre(
            g, r, sb, xc,
            p['tc_Wv'][li], p['tc_bv'][li].reshape(1, F),
            wih0[:, :F].T, wih0[:, F:].T, b0)

        b1s = (p['lstm_bih1'][li] + p['lstm_bhh1'][li]).reshape(1, G4)
        wc = jnp.concatenate(
            [p['lstm_Wih1'][li].T, p['lstm_Whh1'][li].T], axis=0)
        xc = _lstm(pre, p['lstm_Whh0'][li].T, wc, b1s,
                   relu=(li in (0, 2)))

    h4 = xc
    h = _head(h4, p['cbf_W'], p['cbf_b'])
    return h4, h
